# baseline (speedup 1.0000x reference)
import jax
import jax.numpy as jnp
from jax import lax
from jax.experimental import pallas as pl
from jax.experimental.pallas import tpu as pltpu
from jax.experimental.pallas import tpu_sc as plsc

N_TOK = 32768
D_MODEL = 768
N_EXP = 64
NC = 2
NS = 16
NW = NC * NS
L = 16

H = 2
NTH = N_TOK // H
TPW = NTH // NW
TOK_BLK = 4096
CPB = TOK_BLK // TPW


def _gate_body(x_ref, w_ref, b_ref, o_ref):
    logits = lax.dot_general(
        w_ref[...], x_ref[...],
        (((1,), (1,)), ((), ())),
        preferred_element_type=jnp.float32,
    )
    logits = logits + b_ref[...]
    for c in range(CPB):
        o_ref[c] = logits[:, c * TPW:(c + 1) * TPW]


def _gate(x, W, b2):
    return pl.pallas_call(
        _gate_body,
        grid=(NTH // TOK_BLK,),
        in_specs=[
            pl.BlockSpec((TOK_BLK, D_MODEL), lambda i: (i, 0)),
            pl.BlockSpec((N_EXP, D_MODEL), lambda i: (0, 0)),
            pl.BlockSpec((N_EXP, 1), lambda i: (0, 0)),
        ],
        out_specs=pl.BlockSpec((CPB, N_EXP, TPW), lambda i: (i, 0, 0)),
        out_shape=jax.ShapeDtypeStruct((NW, N_EXP, TPW), jnp.float32),
    )(x, W, b2)


def _topk_body(logits_hbm, out1_hbm, out2_hbm, buf, buf1, buf2):
    cid = lax.axis_index("c")
    sid = lax.axis_index("s")
    wid = sid * NC + cid
    base = wid * TPW

    pltpu.sync_copy(logits_hbm.at[wid], buf)

    zeros16 = jnp.zeros((L,), jnp.int32)
    neg_inf = jnp.full((L,), -jnp.inf, jnp.float32)

    def group(g, carry):
        gb = g * L
        max1 = buf[0, pl.ds(gb, L)]
        idx1 = zeros16
        max2 = neg_inf
        idx2 = zeros16
        for e in range(1, N_EXP):
            v = buf[e, pl.ds(gb, L)]
            e_vec = jnp.full((L,), e, jnp.int32)
            gt1 = v > max1
            gt2 = v > max2
            max2 = jnp.where(gt1, max1, jnp.where(gt2, v, max2))
            idx2 = jnp.where(gt1, idx1, jnp.where(gt2, e_vec, idx2))
            max1 = jnp.where(gt1, v, max1)
            idx1 = jnp.where(gt1, e_vec, idx1)
        buf1[pl.ds(gb, L)] = idx1
        buf2[pl.ds(gb, L)] = idx2
        return carry

    lax.fori_loop(0, TPW // L, group, 0)
    pltpu.sync_copy(buf1, out1_hbm.at[pl.ds(base, TPW)])
    pltpu.sync_copy(buf2, out2_hbm.at[pl.ds(base, TPW)])


def _topk(logits_t):
    mesh = plsc.VectorSubcoreMesh(
        core_axis_name="c", subcore_axis_name="s",
        num_cores=NC, num_subcores=NS,
    )
    return pl.kernel(
        _topk_body,
        out_type=[
            jax.ShapeDtypeStruct((NTH,), jnp.int32),
            jax.ShapeDtypeStruct((NTH,), jnp.int32),
        ],
        mesh=mesh,
        scratch_types=[
            pltpu.VMEM((N_EXP, TPW), jnp.float32),
            pltpu.VMEM((TPW,), jnp.int32),
            pltpu.VMEM((TPW,), jnp.int32),
        ],
    )(logits_t)


@jax.jit
def _run(x, W, b):
    b2 = b.reshape(N_EXP, 1)
    parts = []
    for h in range(H):
        lt = _gate(lax.slice_in_dim(x, h * NTH, (h + 1) * NTH, axis=0), W, b2)
        parts.append(_topk(lt))
    r1 = jnp.concatenate([p[0] for p in parts])
    r2 = jnp.concatenate([p[1] for p in parts])
    return jnp.stack([r1, r2], axis=1)


def kernel(x, W, b, top_k):
    return _run(x, W, b)


# speedup vs baseline: 1.9523x; 1.9523x over previous
import jax
import jax.numpy as jnp
from jax import lax
from jax.experimental import pallas as pl
from jax.experimental.pallas import tpu as pltpu
from jax.experimental.pallas import tpu_sc as plsc

N_TOK = 32768
D_MODEL = 768
N_EXP = 64
NC = 2
NS = 16
NW = NC * NS
L = 16

H = 1
NTH = N_TOK // H
TPW = NTH // NW
TOK_BLK = 4096
CPB = TOK_BLK // TPW


def _gate_body(x_ref, w_ref, b_ref, o_ref):
    logits = lax.dot_general(
        w_ref[...], x_ref[...],
        (((1,), (1,)), ((), ())),
        preferred_element_type=jnp.float32,
    )
    logits = logits + b_ref[...]
    for c in range(CPB):
        o_ref[c] = logits[:, c * TPW:(c + 1) * TPW]


def _gate(x, W, b2):
    return pl.pallas_call(
        _gate_body,
        grid=(NTH // TOK_BLK,),
        in_specs=[
            pl.BlockSpec((TOK_BLK, D_MODEL), lambda i: (i, 0)),
            pl.BlockSpec((N_EXP, D_MODEL), lambda i: (0, 0)),
            pl.BlockSpec((N_EXP, 1), lambda i: (0, 0)),
        ],
        out_specs=pl.BlockSpec((CPB, N_EXP, TPW), lambda i: (i, 0, 0)),
        out_shape=jax.ShapeDtypeStruct((NW, N_EXP, TPW), jnp.float32),
    )(x, W, b2)


def _topk_body(logits_hbm, out1_hbm, out2_hbm, buf, buf1, buf2):
    cid = lax.axis_index("c")
    sid = lax.axis_index("s")
    wid = sid * NC + cid
    base = wid * TPW

    pltpu.sync_copy(logits_hbm.at[wid], buf)

    zeros16 = jnp.zeros((L,), jnp.int32)
    neg_inf = jnp.full((L,), -jnp.inf, jnp.float32)

    def group(g, carry):
        gb = g * L
        max1 = buf[0, pl.ds(gb, L)]
        idx1 = zeros16
        max2 = neg_inf
        idx2 = zeros16
        for e in range(1, N_EXP):
            v = buf[e, pl.ds(gb, L)]
            e_vec = jnp.full((L,), e, jnp.int32)
            gt1 = v > max1
            gt2 = v > max2
            max2 = jnp.where(gt1, max1, jnp.where(gt2, v, max2))
            idx2 = jnp.where(gt1, idx1, jnp.where(gt2, e_vec, idx2))
            max1 = jnp.where(gt1, v, max1)
            idx1 = jnp.where(gt1, e_vec, idx1)
        buf1[pl.ds(gb, L)] = idx1
        buf2[pl.ds(gb, L)] = idx2
        return carry

    lax.fori_loop(0, TPW // L, group, 0)
    pltpu.sync_copy(buf1, out1_hbm.at[pl.ds(base, TPW)])
    pltpu.sync_copy(buf2, out2_hbm.at[pl.ds(base, TPW)])


def _topk(logits_t):
    mesh = plsc.VectorSubcoreMesh(
        core_axis_name="c", subcore_axis_name="s",
        num_cores=NC, num_subcores=NS,
    )
    return pl.kernel(
        _topk_body,
        out_type=[
            jax.ShapeDtypeStruct((NTH,), jnp.int32),
            jax.ShapeDtypeStruct((NTH,), jnp.int32),
        ],
        mesh=mesh,
        scratch_types=[
            pltpu.VMEM((N_EXP, TPW), jnp.float32),
            pltpu.VMEM((TPW,), jnp.int32),
            pltpu.VMEM((TPW,), jnp.int32),
        ],
    )(logits_t)


@jax.jit
def _run(x, W, b):
    b2 = b.reshape(N_EXP, 1)
    parts = []
    for h in range(H):
        lt = _gate(lax.slice_in_dim(x, h * NTH, (h + 1) * NTH, axis=0), W, b2)
        parts.append(_topk(lt))
    r1 = jnp.concatenate([p[0] for p in parts])
    r2 = jnp.concatenate([p[1] for p in parts])
    return jnp.stack([r1, r2], axis=1)


def kernel(x, W, b, top_k):
    return _run(x, W, b)
